# Initial kernel scaffold; baseline (speedup 1.0000x reference)
import jax
import jax.numpy as jnp
from jax.experimental import pallas as pl


def kernel(H, reg_mask_prev, reg_coords, W_edge, K_edge, Q_lin, raw_P_edge, W_reg, beta_cos, beta_sin):
    raise NotImplementedError("write your pallas kernel here")


# speedup vs baseline: 15.1693x; 15.1693x over previous
import math

import jax
import jax.numpy as jnp
from jax.experimental import pallas as pl

R = 64
D = 256
M_REG = 8
N_NB = 6
FB_ALPHA = 0.1
FB_SCALE = 1.0 / math.sqrt(M_REG)
NB_OFFS = (1, 63, 8, 56, 9, 55)
INV_SQRT_D = 1.0 / math.sqrt(D)


def _router_kernel(h_ref, htc_ref, coords_ref, mask_ref, w_ref, k_ref,
                   q_ref, pt_ref, wreg_ref, bcos_ref, bsin_ref, out_ref):
    r = pl.program_id(0)

    onehot_r = (jax.lax.broadcasted_iota(jnp.int32, (1, R), 1) == r
                ).astype(jnp.float32)
    hr_row = h_ref[pl.ds(r, 1), :]
    hr_col = jnp.sum(htc_ref[...] * onehot_r, axis=1, keepdims=True)
    coords_r = coords_ref[pl.ds(r, 1), :]

    q2 = q_ref[0]
    q_col = jnp.sum(q2 * hr_row, axis=1, keepdims=True)

    wreg = wreg_ref[...]
    bcos = bcos_ref[...]
    bsin = bsin_ref[...]

    msgs = []
    scores = []
    robust = []
    masks = []
    for j, off in enumerate(NB_OFFS):
        idx = jax.lax.rem(r + off, R)
        hs = h_ref[pl.ds(idx, 1), :]

        msg = jnp.sum(w_ref[j] * hs, axis=1, keepdims=True)
        kcol = jnp.sum(k_ref[j] * hs, axis=1, keepdims=True)

        coords_s = coords_ref[pl.ds(idx, 1), :]
        delta = coords_r - coords_s
        phase = jnp.sum(wreg * delta, axis=1, keepdims=True)
        b = FB_SCALE * (jnp.sum(jnp.cos(phase) * bcos, keepdims=True)
                        + jnp.sum(jnp.sin(phase) * bsin, keepdims=True))
        msg = (1.0 + FB_ALPHA * b) * msg

        score = jnp.sum(q_col * kcol, keepdims=True) * INV_SQRT_D

        resid = msg - hr_col
        p_col = jax.nn.softplus(pt_ref[0, :, j:j + 1])
        mah = jnp.sum(resid * resid * p_col, keepdims=True)
        w_rob = jnp.exp(-0.5 * mah)

        msgs.append(msg)
        scores.append(score)
        robust.append(w_rob)
        masks.append(mask_ref[pl.ds(idx, 1), :])

    neg_inf = jnp.float32(-jnp.inf)
    s_masked = [jnp.where(m > 0, s, neg_inf) for m, s in zip(masks, scores)]
    any_m = masks[0]
    for m in masks[1:]:
        any_m = jnp.maximum(any_m, m)
    mx = s_masked[0]
    for s in s_masked[1:]:
        mx = jnp.maximum(mx, s)
    mx = jnp.where(any_m > 0, mx, 0.0)
    unn = [jnp.exp(s - mx) for s in s_masked]
    total = unn[0]
    for u in unn[1:]:
        total = total + u
    denom = jnp.where(any_m > 0, total, 1.0)
    w = [(u / denom) * rb for u, rb in zip(unn, robust)]
    z = w[0]
    for t in w[1:]:
        z = z + t
    w = [jnp.where(z > 0, t / z, t) for t in w]

    acc = w[0] * msgs[0]
    for t, m in zip(w[1:], msgs[1:]):
        acc = acc + t * m

    out_ref[...] = jnp.where(onehot_r > 0, acc, out_ref[...])


def kernel(H, reg_mask_prev, reg_coords, W_edge, K_edge, Q_lin, raw_P_edge,
           W_reg, beta_cos, beta_sin):
    HT = H.T
    mask_f = reg_mask_prev.astype(jnp.float32).reshape(R, 1)
    PT = raw_P_edge.reshape(R, N_NB, D).transpose(0, 2, 1)
    bcos = beta_cos.reshape(M_REG, 1)
    bsin = beta_sin.reshape(M_REG, 1)

    out_t = pl.pallas_call(
        _router_kernel,
        grid=(R,),
        in_specs=[
            pl.BlockSpec((R, D), lambda r: (0, 0)),
            pl.BlockSpec((D, R), lambda r: (0, 0)),
            pl.BlockSpec((R, 2), lambda r: (0, 0)),
            pl.BlockSpec((R, 1), lambda r: (0, 0)),
            pl.BlockSpec((N_NB, D, D), lambda r: (r, 0, 0)),
            pl.BlockSpec((N_NB, D, D), lambda r: (r, 0, 0)),
            pl.BlockSpec((1, D, D), lambda r: (r, 0, 0)),
            pl.BlockSpec((1, D, N_NB), lambda r: (r, 0, 0)),
            pl.BlockSpec((M_REG, 2), lambda r: (0, 0)),
            pl.BlockSpec((M_REG, 1), lambda r: (0, 0)),
            pl.BlockSpec((M_REG, 1), lambda r: (0, 0)),
        ],
        out_specs=pl.BlockSpec((D, R), lambda r: (0, 0)),
        out_shape=jax.ShapeDtypeStruct((D, R), jnp.float32),
    )(H, HT, reg_coords, mask_f, W_edge, K_edge, Q_lin, PT,
      W_reg, bcos, bsin)
    return out_t.T
